# baseline (speedup 1.0000x reference)
import functools

import jax
import jax.numpy as jnp
from jax import lax
from jax.experimental import pallas as pl
from jax.experimental.pallas import tpu as pltpu
from jax.experimental.pallas import tpu_sc as plsc


@functools.lru_cache(maxsize=None)
def _make_sc_gather(n_rows: int, d: int):
  info = plsc.get_sparse_core_info()
  nw = info.num_cores * info.num_subcores
  chunk = 8
  assert n_rows % chunk == 0
  n_chunks = n_rows // chunk
  n_extra = n_chunks - nw
  assert 0 <= n_extra <= nw
  mesh = plsc.VectorSubcoreMesh(core_axis_name="c", subcore_axis_name="s")

  @functools.partial(
      pl.kernel,
      mesh=mesh,
      out_type=jax.ShapeDtypeStruct((n_rows, d), jnp.float32),
      scratch_types=[
          pltpu.VMEM((chunk,), jnp.int32),
          pltpu.VMEM((chunk, d), jnp.float32),
          pltpu.SemaphoreType.DMA,
      ],
  )
  def gather_kernel(table_hbm, idx_hbm, out_hbm, idx_v, rows_v, sem):
    wid = lax.axis_index("s") * info.num_cores + lax.axis_index("c")

    def do_chunk(cid):
      base = pl.multiple_of(cid * chunk, chunk)
      pltpu.sync_copy(idx_hbm.at[pl.ds(base, chunk)], idx_v)
      pltpu.async_copy(table_hbm.at[idx_v], rows_v, sem).wait()
      pltpu.sync_copy(rows_v, out_hbm.at[pl.ds(base, chunk)])

    do_chunk(wid)
    if n_extra:
      @pl.when(wid < n_extra)
      def _():
        do_chunk(wid + nw)

  return gather_kernel


def _recurrence_body(emb_ref, enc_ref, v2h_W_ref, v2h_b_ref, att_Wh_ref,
                     att_We_ref, att_v_ref, W_ih_ref, W_hh_ref, b_ih_ref,
                     b_hh_ref, hid_ref, attn_ref):
  enc = enc_ref[...]
  b, f, h_dim = enc.shape
  t_steps = emb_ref.shape[0]

  mean_v = jnp.mean(enc, axis=1)
  h = jnp.tanh(
      jnp.dot(mean_v, v2h_W_ref[...], preferred_element_type=jnp.float32)
      + v2h_b_ref[...])
  e_proj = jnp.dot(enc.reshape(b * f, h_dim), att_We_ref[...],
                   preferred_element_type=jnp.float32).reshape(b, f, h_dim)

  att_Wh = att_Wh_ref[...]
  att_v = att_v_ref[...]
  W_ih = W_ih_ref[...]
  W_hh = W_hh_ref[...]
  b_ih = b_ih_ref[...]
  b_hh = b_hh_ref[...]

  for i in range(t_steps):
    emb_i = emb_ref[i]
    hw = jnp.dot(h, att_Wh, preferred_element_type=jnp.float32)
    tt = jnp.tanh(hw[:, None, :] + e_proj)
    scores = jnp.sum(tt * att_v[None, None, :], axis=-1)
    m = jnp.max(scores, axis=-1, keepdims=True)
    e = jnp.exp(scores - m)
    probs = e / jnp.sum(e, axis=-1, keepdims=True)
    ctx = jnp.sum(probs[:, :, None] * enc, axis=1)

    gi = (jnp.dot(emb_i, W_ih[:h_dim], preferred_element_type=jnp.float32)
          + jnp.dot(ctx, W_ih[h_dim:], preferred_element_type=jnp.float32)
          + b_ih)
    gh = jnp.dot(h, W_hh, preferred_element_type=jnp.float32) + b_hh
    r = jax.nn.sigmoid(gi[:, :h_dim] + gh[:, :h_dim])
    z = jax.nn.sigmoid(gi[:, h_dim:2 * h_dim] + gh[:, h_dim:2 * h_dim])
    n = jnp.tanh(gi[:, 2 * h_dim:] + r * gh[:, 2 * h_dim:])
    h = (1.0 - z) * n + z * h

    hid_ref[i] = h
    attn_ref[i] = probs


_NBUF = 4
_OBUF = 4


def _make_logits_manual(n_rows, h_dim, v, vt):
  n_full, rem = divmod(v, vt)
  widths = [vt] * n_full + ([rem] if rem else [])
  n_tiles = len(widths)

  def body(h_ref, b_ref, w_hbm, out_hbm, w_bufs, o_bufs, w_tail, o_tail,
           in_sems, out_sems, tail_in_sem, tail_out_sem):
    h16 = h_ref[...].astype(jnp.bfloat16)

    def in_copy(j):
      if rem and j == n_tiles - 1:
        return pltpu.make_async_copy(
            w_hbm.at[:, pl.ds(j * vt, rem)], w_tail, tail_in_sem)
      return pltpu.make_async_copy(
          w_hbm.at[:, pl.ds(j * vt, vt)], w_bufs.at[j % _NBUF],
          in_sems.at[j % _NBUF])

    def out_copy(j):
      if rem and j == n_tiles - 1:
        return pltpu.make_async_copy(
            o_tail, out_hbm.at[:, pl.ds(j * vt, rem)], tail_out_sem)
      return pltpu.make_async_copy(
          o_bufs.at[j % _OBUF], out_hbm.at[:, pl.ds(j * vt, vt)],
          out_sems.at[j % _OBUF])

    for j in range(min(_NBUF, n_tiles)):
      in_copy(j).start()
    for j in range(n_tiles):
      is_tail = rem and j == n_tiles - 1
      in_copy(j).wait()
      if j >= _OBUF:
        out_copy(j - _OBUF).wait()
      w_now = w_tail if is_tail else w_bufs[j % _NBUF]
      acc = jnp.dot(h16, w_now[...].astype(jnp.bfloat16)
                    if is_tail else w_now.astype(jnp.bfloat16),
                    preferred_element_type=jnp.float32)
      res = acc + b_ref[:, j * vt:j * vt + widths[j]]
      if is_tail:
        o_tail[...] = res
      else:
        o_bufs[j % _OBUF] = res
      out_copy(j).start()
      if j + _NBUF < n_tiles:
        in_copy(j + _NBUF).start()
    for j in range(max(n_tiles - _OBUF, 0), n_tiles):
      out_copy(j).wait()

  return pl.pallas_call(
      body,
      in_specs=[
          pl.BlockSpec(memory_space=pltpu.VMEM),
          pl.BlockSpec(memory_space=pltpu.VMEM),
          pl.BlockSpec(memory_space=pl.ANY),
      ],
      out_specs=pl.BlockSpec(memory_space=pl.ANY),
      out_shape=jax.ShapeDtypeStruct((n_rows, v), jnp.float32),
      scratch_shapes=[
          pltpu.VMEM((_NBUF, h_dim, vt), jnp.float32),
          pltpu.VMEM((_OBUF, n_rows, vt), jnp.float32),
          pltpu.VMEM((h_dim, rem if rem else 128), jnp.float32),
          pltpu.VMEM((n_rows, rem if rem else 128), jnp.float32),
          pltpu.SemaphoreType.DMA((_NBUF,)),
          pltpu.SemaphoreType.DMA((_OBUF,)),
          pltpu.SemaphoreType.DMA,
          pltpu.SemaphoreType.DMA,
      ],
  )


def kernel(input_ids, encoder_hidden_states, embedding, v2h_W, v2h_b,
           att_Wh, att_We, att_v, W_ih, W_hh, b_ih, b_hh, cls_W, cls_b):
  b, t = input_ids.shape
  _, f, h_dim = encoder_hidden_states.shape
  v = cls_W.shape[1]

  flat_ids = input_ids.T.reshape(-1).astype(jnp.int32)
  rows_tb = _make_sc_gather(t * b, h_dim)(embedding, flat_ids)
  emb_tbh = rows_tb.reshape(t, b, h_dim)
  sentence_embs = jnp.transpose(emb_tbh, (1, 0, 2))

  hid_tbh, attn_tbf = pl.pallas_call(
      _recurrence_body,
      out_shape=(
          jax.ShapeDtypeStruct((t, b, h_dim), jnp.float32),
          jax.ShapeDtypeStruct((t, b, f), jnp.float32),
      ),
  )(emb_tbh, encoder_hidden_states, v2h_W, v2h_b, att_Wh, att_We,
    att_v, W_ih, W_hh, b_ih, b_hh)

  out_hidden = jnp.transpose(hid_tbh, (1, 0, 2))
  out_attn = jnp.transpose(attn_tbf, (1, 2, 0))

  vt = 2048
  hidden_flat = hid_tbh.reshape(t * b, h_dim)
  logits_flat = _make_logits_manual(t * b, h_dim, v, vt)(
      hidden_flat, cls_b.reshape(1, v), cls_W)
  out_logits = jnp.transpose(logits_flat.reshape(t, b, v), (1, 0, 2))

  return out_hidden, out_attn, out_logits, sentence_embs


# speedup vs baseline: 2.0131x; 2.0131x over previous
import functools

import jax
import jax.numpy as jnp
from jax import lax
from jax.experimental import pallas as pl
from jax.experimental.pallas import tpu as pltpu
from jax.experimental.pallas import tpu_sc as plsc


@functools.lru_cache(maxsize=None)
def _make_sc_gather(n_rows: int, d: int):
  info = plsc.get_sparse_core_info()
  nw = info.num_cores * info.num_subcores
  chunk = 8
  assert n_rows % chunk == 0
  n_chunks = n_rows // chunk
  n_extra = n_chunks - nw
  assert 0 <= n_extra <= nw
  mesh = plsc.VectorSubcoreMesh(core_axis_name="c", subcore_axis_name="s")

  @functools.partial(
      pl.kernel,
      mesh=mesh,
      out_type=jax.ShapeDtypeStruct((n_rows, d), jnp.float32),
      scratch_types=[
          pltpu.VMEM((chunk,), jnp.int32),
          pltpu.VMEM((chunk, d), jnp.float32),
          pltpu.SemaphoreType.DMA,
      ],
  )
  def gather_kernel(table_hbm, idx_hbm, out_hbm, idx_v, rows_v, sem):
    wid = lax.axis_index("s") * info.num_cores + lax.axis_index("c")

    def do_chunk(cid):
      base = pl.multiple_of(cid * chunk, chunk)
      pltpu.sync_copy(idx_hbm.at[pl.ds(base, chunk)], idx_v)
      pltpu.async_copy(table_hbm.at[idx_v], rows_v, sem).wait()
      pltpu.sync_copy(rows_v, out_hbm.at[pl.ds(base, chunk)])

    do_chunk(wid)
    if n_extra:
      @pl.when(wid < n_extra)
      def _():
        do_chunk(wid + nw)

  return gather_kernel


def _recurrence_body(emb_ref, enc_ref, v2h_W_ref, v2h_b_ref, att_Wh_ref,
                     att_We_ref, att_v_ref, W_ih_ref, W_hh_ref, b_ih_ref,
                     b_hh_ref, hid_ref, attn_ref):
  enc = enc_ref[...]
  b, f, h_dim = enc.shape
  t_steps = emb_ref.shape[0]

  mean_v = jnp.mean(enc, axis=1)
  h = jnp.tanh(
      jnp.dot(mean_v, v2h_W_ref[...], preferred_element_type=jnp.float32)
      + v2h_b_ref[...])
  e_proj = jnp.dot(enc.reshape(b * f, h_dim), att_We_ref[...],
                   preferred_element_type=jnp.float32).reshape(b, f, h_dim)

  att_Wh = att_Wh_ref[...]
  att_v = att_v_ref[...]
  W_ih = W_ih_ref[...]
  W_hh = W_hh_ref[...]
  b_ih = b_ih_ref[...]
  b_hh = b_hh_ref[...]

  for i in range(t_steps):
    emb_i = emb_ref[i]
    hw = jnp.dot(h, att_Wh, preferred_element_type=jnp.float32)
    tt = jnp.tanh(hw[:, None, :] + e_proj)
    scores = jnp.sum(tt * att_v[None, None, :], axis=-1)
    m = jnp.max(scores, axis=-1, keepdims=True)
    e = jnp.exp(scores - m)
    probs = e / jnp.sum(e, axis=-1, keepdims=True)
    ctx = jnp.sum(probs[:, :, None] * enc, axis=1)

    gi = (jnp.dot(emb_i, W_ih[:h_dim], preferred_element_type=jnp.float32)
          + jnp.dot(ctx, W_ih[h_dim:], preferred_element_type=jnp.float32)
          + b_ih)
    gh = jnp.dot(h, W_hh, preferred_element_type=jnp.float32) + b_hh
    r = jax.nn.sigmoid(gi[:, :h_dim] + gh[:, :h_dim])
    z = jax.nn.sigmoid(gi[:, h_dim:2 * h_dim] + gh[:, h_dim:2 * h_dim])
    n = jnp.tanh(gi[:, 2 * h_dim:] + r * gh[:, 2 * h_dim:])
    h = (1.0 - z) * n + z * h

    hid_ref[i] = h
    attn_ref[i] = probs


_NBUF = 4
_OBUF = 4


def _make_logits_manual(n_rows, h_dim, v, vt):
  n_full, rem = divmod(v, vt)
  widths = [vt] * n_full + ([rem] if rem else [])
  n_tiles = len(widths)

  def body(h_ref, b_ref, wt_hbm, out_hbm, w_bufs, o_bufs, o_tail,
           in_sems, out_sems, tail_out_sem):
    h16 = h_ref[...].astype(jnp.bfloat16)

    def in_copy(j):
      return pltpu.make_async_copy(
          wt_hbm.at[pl.ds(j * vt, widths[j]), :],
          w_bufs.at[j % _NBUF, pl.ds(0, widths[j]), :],
          in_sems.at[j % _NBUF])

    def out_copy(j):
      if rem and j == n_tiles - 1:
        return pltpu.make_async_copy(
            o_tail, out_hbm.at[:, pl.ds(j * vt, rem)], tail_out_sem)
      return pltpu.make_async_copy(
          o_bufs.at[j % _OBUF], out_hbm.at[:, pl.ds(j * vt, vt)],
          out_sems.at[j % _OBUF])

    for j in range(min(_NBUF, n_tiles)):
      in_copy(j).start()
    for j in range(n_tiles):
      is_tail = rem and j == n_tiles - 1
      wt = widths[j]
      in_copy(j).wait()
      if j >= _OBUF:
        out_copy(j - _OBUF).wait()
      w16 = w_bufs[j % _NBUF, :wt, :].astype(jnp.bfloat16)
      acc = lax.dot_general(h16, w16, (((1,), (1,)), ((), ())),
                            preferred_element_type=jnp.float32)
      res = acc + b_ref[:, j * vt:j * vt + wt]
      if is_tail:
        o_tail[...] = res
      else:
        o_bufs[j % _OBUF] = res
      out_copy(j).start()
      if j + _NBUF < n_tiles:
        in_copy(j + _NBUF).start()
    for j in range(max(n_tiles - _OBUF, 0), n_tiles):
      out_copy(j).wait()

  return pl.pallas_call(
      body,
      in_specs=[
          pl.BlockSpec(memory_space=pltpu.VMEM),
          pl.BlockSpec(memory_space=pltpu.VMEM),
          pl.BlockSpec(memory_space=pl.ANY),
      ],
      out_specs=pl.BlockSpec(memory_space=pl.ANY),
      out_shape=jax.ShapeDtypeStruct((n_rows, v), jnp.float32),
      scratch_shapes=[
          pltpu.VMEM((_NBUF, vt, h_dim), jnp.float32),
          pltpu.VMEM((_OBUF, n_rows, vt), jnp.float32),
          pltpu.VMEM((n_rows, rem if rem else 128), jnp.float32),
          pltpu.SemaphoreType.DMA((_NBUF,)),
          pltpu.SemaphoreType.DMA((_OBUF,)),
          pltpu.SemaphoreType.DMA,
      ],
  )


def kernel(input_ids, encoder_hidden_states, embedding, v2h_W, v2h_b,
           att_Wh, att_We, att_v, W_ih, W_hh, b_ih, b_hh, cls_W, cls_b):
  b, t = input_ids.shape
  _, f, h_dim = encoder_hidden_states.shape
  v = cls_W.shape[1]

  flat_ids = input_ids.T.reshape(-1).astype(jnp.int32)
  rows_tb = _make_sc_gather(t * b, h_dim)(embedding, flat_ids)
  emb_tbh = rows_tb.reshape(t, b, h_dim)
  sentence_embs = jnp.transpose(emb_tbh, (1, 0, 2))

  hid_tbh, attn_tbf = pl.pallas_call(
      _recurrence_body,
      out_shape=(
          jax.ShapeDtypeStruct((t, b, h_dim), jnp.float32),
          jax.ShapeDtypeStruct((t, b, f), jnp.float32),
      ),
  )(emb_tbh, encoder_hidden_states, v2h_W, v2h_b, att_Wh, att_We,
    att_v, W_ih, W_hh, b_ih, b_hh)

  out_hidden = jnp.transpose(hid_tbh, (1, 0, 2))
  out_attn = jnp.transpose(attn_tbf, (1, 2, 0))

  vt = 2048
  hidden_flat = hid_tbh.reshape(t * b, h_dim)
  logits_flat = _make_logits_manual(t * b, h_dim, v, vt)(
      hidden_flat, cls_b.reshape(1, v), cls_W.T)
  out_logits = jnp.transpose(logits_flat.reshape(t, b, v), (1, 0, 2))

  return out_hidden, out_attn, out_logits, sentence_embs


# speedup vs baseline: 2.0163x; 1.0016x over previous
import functools

import jax
import jax.numpy as jnp
from jax import lax
from jax.experimental import pallas as pl
from jax.experimental.pallas import tpu as pltpu
from jax.experimental.pallas import tpu_sc as plsc


@functools.lru_cache(maxsize=None)
def _make_sc_gather(n_rows: int, d: int):
  info = plsc.get_sparse_core_info()
  nw = info.num_cores * info.num_subcores
  chunk = 8
  assert n_rows % chunk == 0
  n_chunks = n_rows // chunk
  n_extra = n_chunks - nw
  assert 0 <= n_extra <= nw
  mesh = plsc.VectorSubcoreMesh(core_axis_name="c", subcore_axis_name="s")

  @functools.partial(
      pl.kernel,
      mesh=mesh,
      out_type=jax.ShapeDtypeStruct((n_rows, d), jnp.float32),
      scratch_types=[
          pltpu.VMEM((chunk,), jnp.int32),
          pltpu.VMEM((chunk, d), jnp.float32),
          pltpu.SemaphoreType.DMA,
      ],
  )
  def gather_kernel(table_hbm, idx_hbm, out_hbm, idx_v, rows_v, sem):
    wid = lax.axis_index("s") * info.num_cores + lax.axis_index("c")

    def do_chunk(cid):
      base = pl.multiple_of(cid * chunk, chunk)
      pltpu.sync_copy(idx_hbm.at[pl.ds(base, chunk)], idx_v)
      pltpu.async_copy(table_hbm.at[idx_v], rows_v, sem).wait()
      pltpu.sync_copy(rows_v, out_hbm.at[pl.ds(base, chunk)])

    do_chunk(wid)
    if n_extra:
      @pl.when(wid < n_extra)
      def _():
        do_chunk(wid + nw)

  return gather_kernel


def _recurrence_body(emb_ref, enc_ref, v2h_W_ref, v2h_b_ref, att_Wh_ref,
                     att_We_ref, att_v_ref, W_ih_ref, W_hh_ref, b_ih_ref,
                     b_hh_ref, hid_ref, attn_ref):
  enc = enc_ref[...]
  b, f, h_dim = enc.shape
  t_steps = emb_ref.shape[0]

  mean_v = jnp.mean(enc, axis=1)
  h = jnp.tanh(
      jnp.dot(mean_v, v2h_W_ref[...], preferred_element_type=jnp.float32)
      + v2h_b_ref[...])
  e_proj = jnp.dot(enc.reshape(b * f, h_dim), att_We_ref[...],
                   preferred_element_type=jnp.float32).reshape(b, f, h_dim)

  att_Wh = att_Wh_ref[...]
  att_v = att_v_ref[...]
  W_ih = W_ih_ref[...]
  W_hh = W_hh_ref[...]
  b_ih = b_ih_ref[...]
  b_hh = b_hh_ref[...]

  for i in range(t_steps):
    emb_i = emb_ref[i]
    hw = jnp.dot(h, att_Wh, preferred_element_type=jnp.float32)
    tt = jnp.tanh(hw[:, None, :] + e_proj)
    scores = jnp.sum(tt * att_v[None, None, :], axis=-1)
    m = jnp.max(scores, axis=-1, keepdims=True)
    e = jnp.exp(scores - m)
    probs = e / jnp.sum(e, axis=-1, keepdims=True)
    ctx = jnp.sum(probs[:, :, None] * enc, axis=1)

    gi = (jnp.dot(emb_i, W_ih[:h_dim], preferred_element_type=jnp.float32)
          + jnp.dot(ctx, W_ih[h_dim:], preferred_element_type=jnp.float32)
          + b_ih)
    gh = jnp.dot(h, W_hh, preferred_element_type=jnp.float32) + b_hh
    r = jax.nn.sigmoid(gi[:, :h_dim] + gh[:, :h_dim])
    z = jax.nn.sigmoid(gi[:, h_dim:2 * h_dim] + gh[:, h_dim:2 * h_dim])
    n = jnp.tanh(gi[:, 2 * h_dim:] + r * gh[:, 2 * h_dim:])
    h = (1.0 - z) * n + z * h

    hid_ref[i] = h
    attn_ref[i] = probs


_NBUF = 4
_OBUF = 4


def _make_logits_manual(n_rows, h_dim, v, vt):
  n_full, rem = divmod(v, vt)
  widths = [vt] * n_full + ([rem] if rem else [])
  n_tiles = len(widths)

  def body(h_ref, b_ref, wt_hbm, out_hbm, w_bufs, o_bufs, o_tail,
           in_sems, out_sems, tail_out_sem):
    h16 = h_ref[...].astype(jnp.bfloat16)

    def in_copy(j):
      return pltpu.make_async_copy(
          wt_hbm.at[pl.ds(j * vt, widths[j]), :],
          w_bufs.at[j % _NBUF, pl.ds(0, widths[j]), :],
          in_sems.at[j % _NBUF])

    def out_copy(j):
      if rem and j == n_tiles - 1:
        return pltpu.make_async_copy(
            o_tail, out_hbm.at[:, pl.ds(j * vt, rem)], tail_out_sem)
      return pltpu.make_async_copy(
          o_bufs.at[j % _OBUF], out_hbm.at[:, pl.ds(j * vt, vt)],
          out_sems.at[j % _OBUF])

    for j in range(min(_NBUF, n_tiles)):
      in_copy(j).start()
    for j in range(n_tiles):
      is_tail = rem and j == n_tiles - 1
      wt = widths[j]
      in_copy(j).wait()
      if j >= _OBUF:
        out_copy(j - _OBUF).wait()
      w16 = w_bufs[j % _NBUF, :wt, :].astype(jnp.bfloat16)
      acc = lax.dot_general(h16, w16, (((1,), (1,)), ((), ())),
                            preferred_element_type=jnp.float32)
      res = acc + b_ref[:, j * vt:j * vt + wt]
      if is_tail:
        o_tail[...] = res
      else:
        o_bufs[j % _OBUF] = res
      out_copy(j).start()
      if j + _NBUF < n_tiles:
        in_copy(j + _NBUF).start()
    for j in range(max(n_tiles - _OBUF, 0), n_tiles):
      out_copy(j).wait()

  return pl.pallas_call(
      body,
      in_specs=[
          pl.BlockSpec(memory_space=pltpu.VMEM),
          pl.BlockSpec(memory_space=pltpu.VMEM),
          pl.BlockSpec(memory_space=pl.ANY),
      ],
      out_specs=pl.BlockSpec(memory_space=pl.ANY),
      out_shape=jax.ShapeDtypeStruct((n_rows, v), jnp.float32),
      scratch_shapes=[
          pltpu.VMEM((_NBUF, vt, h_dim), jnp.float32),
          pltpu.VMEM((_OBUF, n_rows, vt), jnp.float32),
          pltpu.VMEM((n_rows, rem if rem else 128), jnp.float32),
          pltpu.SemaphoreType.DMA((_NBUF,)),
          pltpu.SemaphoreType.DMA((_OBUF,)),
          pltpu.SemaphoreType.DMA,
      ],
  )


def kernel(input_ids, encoder_hidden_states, embedding, v2h_W, v2h_b,
           att_Wh, att_We, att_v, W_ih, W_hh, b_ih, b_hh, cls_W, cls_b):
  b, t = input_ids.shape
  _, f, h_dim = encoder_hidden_states.shape
  v = cls_W.shape[1]

  flat_ids = input_ids.T.reshape(-1).astype(jnp.int32)
  rows_tb = _make_sc_gather(t * b, h_dim)(embedding, flat_ids)
  emb_tbh = rows_tb.reshape(t, b, h_dim)
  sentence_embs = jnp.transpose(emb_tbh, (1, 0, 2))

  hid_tbh, attn_tbf = pl.pallas_call(
      _recurrence_body,
      out_shape=(
          jax.ShapeDtypeStruct((t, b, h_dim), jnp.float32),
          jax.ShapeDtypeStruct((t, b, f), jnp.float32),
      ),
  )(emb_tbh, encoder_hidden_states, v2h_W, v2h_b, att_Wh, att_We,
    att_v, W_ih, W_hh, b_ih, b_hh)

  out_hidden = jnp.transpose(hid_tbh, (1, 0, 2))
  out_attn = jnp.transpose(attn_tbf, (1, 2, 0))

  vt = 4096
  hidden_flat = hid_tbh.reshape(t * b, h_dim)
  logits_flat = _make_logits_manual(t * b, h_dim, v, vt)(
      hidden_flat, cls_b.reshape(1, v), cls_W.T)
  out_logits = jnp.transpose(logits_flat.reshape(t, b, v), (1, 0, 2))

  return out_hidden, out_attn, out_logits, sentence_embs


# speedup vs baseline: 2.1169x; 1.0499x over previous
import functools

import jax
import jax.numpy as jnp
from jax import lax
from jax.experimental import pallas as pl
from jax.experimental.pallas import tpu as pltpu
from jax.experimental.pallas import tpu_sc as plsc


@functools.lru_cache(maxsize=None)
def _make_sc_gather(n_rows: int, d: int):
  info = plsc.get_sparse_core_info()
  nw = info.num_cores * info.num_subcores
  chunk = 8
  assert n_rows % chunk == 0
  n_chunks = n_rows // chunk
  n_extra = n_chunks - nw
  assert 0 <= n_extra <= nw
  mesh = plsc.VectorSubcoreMesh(core_axis_name="c", subcore_axis_name="s")

  @functools.partial(
      pl.kernel,
      mesh=mesh,
      out_type=jax.ShapeDtypeStruct((n_rows, d), jnp.float32),
      scratch_types=[
          pltpu.VMEM((chunk,), jnp.int32),
          pltpu.VMEM((chunk, d), jnp.float32),
          pltpu.SemaphoreType.DMA,
      ],
  )
  def gather_kernel(table_hbm, idx_hbm, out_hbm, idx_v, rows_v, sem):
    wid = lax.axis_index("s") * info.num_cores + lax.axis_index("c")

    def do_chunk(cid):
      base = pl.multiple_of(cid * chunk, chunk)
      pltpu.sync_copy(idx_hbm.at[pl.ds(base, chunk)], idx_v)
      pltpu.async_copy(table_hbm.at[idx_v], rows_v, sem).wait()
      pltpu.sync_copy(rows_v, out_hbm.at[pl.ds(base, chunk)])

    do_chunk(wid)
    if n_extra:
      @pl.when(wid < n_extra)
      def _():
        do_chunk(wid + nw)

  return gather_kernel


_NBUF = 6
_OBUF = 3


def _make_fused(t_steps, b, f, h_dim, v, vt):
  n_full, rem = divmod(v, vt)
  widths = [vt] * n_full + ([rem] if rem else [])
  n_tiles = len(widths)
  n_rows = t_steps * b

  def body(emb_ref, enc_ref, v2h_W_ref, v2h_b_ref, att_Wh_ref, att_We_ref,
           att_v_ref, W_ih_ref, W_hh_ref, b_ih_ref, b_hh_ref, cb_ref,
           wt_hbm, hid_ref, attn_ref, out_hbm,
           w_bufs, o_bufs, o_tail, in_sems, out_sems, tail_out_sem):

    def in_copy(j):
      return pltpu.make_async_copy(
          wt_hbm.at[pl.ds(j * vt, widths[j]), :],
          w_bufs.at[j % _NBUF, pl.ds(0, widths[j]), :],
          in_sems.at[j % _NBUF])

    def out_copy(j):
      if rem and j == n_tiles - 1:
        return pltpu.make_async_copy(
            o_tail, out_hbm.at[:, pl.ds(j * vt, rem)], tail_out_sem)
      return pltpu.make_async_copy(
          o_bufs.at[j % _OBUF], out_hbm.at[:, pl.ds(j * vt, vt)],
          out_sems.at[j % _OBUF])

    for j in range(min(_NBUF, n_tiles)):
      in_copy(j).start()

    enc = enc_ref[...]
    mean_v = jnp.mean(enc, axis=1)
    h = jnp.tanh(
        jnp.dot(mean_v, v2h_W_ref[...], preferred_element_type=jnp.float32)
        + v2h_b_ref[...])
    e_proj = jnp.dot(enc.reshape(b * f, h_dim), att_We_ref[...],
                     preferred_element_type=jnp.float32).reshape(b, f, h_dim)

    att_Wh = att_Wh_ref[...]
    att_v = att_v_ref[...]
    W_ih = W_ih_ref[...]
    W_hh = W_hh_ref[...]
    b_ih = b_ih_ref[...]
    b_hh = b_hh_ref[...]

    for i in range(t_steps):
      emb_i = emb_ref[i]
      hw = jnp.dot(h, att_Wh, preferred_element_type=jnp.float32)
      tt = jnp.tanh(hw[:, None, :] + e_proj)
      scores = jnp.sum(tt * att_v[None, None, :], axis=-1)
      m = jnp.max(scores, axis=-1, keepdims=True)
      e = jnp.exp(scores - m)
      probs = e / jnp.sum(e, axis=-1, keepdims=True)
      ctx = jnp.sum(probs[:, :, None] * enc, axis=1)

      gi = (jnp.dot(emb_i, W_ih[:h_dim], preferred_element_type=jnp.float32)
            + jnp.dot(ctx, W_ih[h_dim:], preferred_element_type=jnp.float32)
            + b_ih)
      gh = jnp.dot(h, W_hh, preferred_element_type=jnp.float32) + b_hh
      r = jax.nn.sigmoid(gi[:, :h_dim] + gh[:, :h_dim])
      z = jax.nn.sigmoid(gi[:, h_dim:2 * h_dim] + gh[:, h_dim:2 * h_dim])
      n = jnp.tanh(gi[:, 2 * h_dim:] + r * gh[:, 2 * h_dim:])
      h = (1.0 - z) * n + z * h

      hid_ref[i] = h
      attn_ref[i] = probs

    h16 = hid_ref[...].reshape(n_rows, h_dim).astype(jnp.bfloat16)
    for j in range(n_tiles):
      is_tail = rem and j == n_tiles - 1
      wt = widths[j]
      in_copy(j).wait()
      if j >= _OBUF:
        out_copy(j - _OBUF).wait()
      w16 = w_bufs[j % _NBUF, :wt, :].astype(jnp.bfloat16)
      acc = lax.dot_general(h16, w16, (((1,), (1,)), ((), ())),
                            preferred_element_type=jnp.float32)
      res = acc + cb_ref[:, j * vt:j * vt + wt]
      if is_tail:
        o_tail[...] = res
      else:
        o_bufs[j % _OBUF] = res
      out_copy(j).start()
      if j + _NBUF < n_tiles:
        in_copy(j + _NBUF).start()
    for j in range(max(n_tiles - _OBUF, 0), n_tiles):
      out_copy(j).wait()

  return pl.pallas_call(
      body,
      in_specs=[pl.BlockSpec(memory_space=pltpu.VMEM)] * 12
      + [pl.BlockSpec(memory_space=pl.ANY)],
      out_specs=(
          pl.BlockSpec(memory_space=pltpu.VMEM),
          pl.BlockSpec(memory_space=pltpu.VMEM),
          pl.BlockSpec(memory_space=pl.ANY),
      ),
      out_shape=(
          jax.ShapeDtypeStruct((t_steps, b, h_dim), jnp.float32),
          jax.ShapeDtypeStruct((t_steps, b, f), jnp.float32),
          jax.ShapeDtypeStruct((n_rows, v), jnp.float32),
      ),
      scratch_shapes=[
          pltpu.VMEM((_NBUF, vt, h_dim), jnp.float32),
          pltpu.VMEM((_OBUF, n_rows, vt), jnp.float32),
          pltpu.VMEM((n_rows, rem if rem else 128), jnp.float32),
          pltpu.SemaphoreType.DMA((_NBUF,)),
          pltpu.SemaphoreType.DMA((_OBUF,)),
          pltpu.SemaphoreType.DMA,
      ],
  )


def kernel(input_ids, encoder_hidden_states, embedding, v2h_W, v2h_b,
           att_Wh, att_We, att_v, W_ih, W_hh, b_ih, b_hh, cls_W, cls_b):
  b, t = input_ids.shape
  _, f, h_dim = encoder_hidden_states.shape
  v = cls_W.shape[1]

  flat_ids = input_ids.T.reshape(-1).astype(jnp.int32)
  rows_tb = _make_sc_gather(t * b, h_dim)(embedding, flat_ids)
  emb_tbh = rows_tb.reshape(t, b, h_dim)
  sentence_embs = jnp.transpose(emb_tbh, (1, 0, 2))

  hid_tbh, attn_tbf, logits_flat = _make_fused(t, b, f, h_dim, v, 2048)(
      emb_tbh, encoder_hidden_states, v2h_W, v2h_b, att_Wh, att_We, att_v,
      W_ih, W_hh, b_ih, b_hh, cls_b.reshape(1, v), cls_W.T)

  out_hidden = jnp.transpose(hid_tbh, (1, 0, 2))
  out_attn = jnp.transpose(attn_tbf, (1, 2, 0))
  out_logits = jnp.transpose(logits_flat.reshape(t, b, v), (1, 0, 2))

  return out_hidden, out_attn, out_logits, sentence_embs


# speedup vs baseline: 2.2012x; 1.0398x over previous
import functools

import jax
import jax.numpy as jnp
from jax import lax
from jax.experimental import pallas as pl
from jax.experimental.pallas import tpu as pltpu
from jax.experimental.pallas import tpu_sc as plsc


@functools.lru_cache(maxsize=None)
def _make_sc_gather(n_rows: int, d: int):
  info = plsc.get_sparse_core_info()
  nw = info.num_cores * info.num_subcores
  chunk = 8
  assert n_rows % chunk == 0
  n_chunks = n_rows // chunk
  n_extra = n_chunks - nw
  assert 0 <= n_extra <= nw
  mesh = plsc.VectorSubcoreMesh(core_axis_name="c", subcore_axis_name="s")

  @functools.partial(
      pl.kernel,
      mesh=mesh,
      out_type=jax.ShapeDtypeStruct((n_rows, d), jnp.float32),
      scratch_types=[
          pltpu.VMEM((chunk,), jnp.int32),
          pltpu.VMEM((chunk, d), jnp.float32),
          pltpu.SemaphoreType.DMA,
      ],
  )
  def gather_kernel(table_hbm, idx_hbm, out_hbm, idx_v, rows_v, sem):
    wid = lax.axis_index("s") * info.num_cores + lax.axis_index("c")

    def do_chunk(cid):
      base = pl.multiple_of(cid * chunk, chunk)
      pltpu.sync_copy(idx_hbm.at[pl.ds(base, chunk)], idx_v)
      pltpu.async_copy(table_hbm.at[idx_v], rows_v, sem).wait()
      pltpu.sync_copy(rows_v, out_hbm.at[pl.ds(base, chunk)])

    do_chunk(wid)
    if n_extra:
      @pl.when(wid < n_extra)
      def _():
        do_chunk(wid + nw)

  return gather_kernel


_NBUF = 6
_OBUF = 3


def _make_fused(t_steps, b, f, h_dim, v, vt):
  n_full, rem = divmod(v, vt)
  widths = [vt] * n_full + ([rem] if rem else [])
  n_tiles = len(widths)
  n_rows = t_steps * b

  def body(emb_ref, enc_ref, v2h_W_ref, v2h_b_ref, att_Wh_ref, att_We_ref,
           att_v_ref, W_ih_ref, W_hh_ref, b_ih_ref, b_hh_ref, cb_ref,
           wt_hbm, hid_ref, attn_ref, out_hbm,
           w_bufs, o_bufs, o_tail, in_sems, out_sems, tail_out_sem):

    def in_copy(j):
      return pltpu.make_async_copy(
          wt_hbm.at[pl.ds(j * vt, widths[j]), :],
          w_bufs.at[j % _NBUF, pl.ds(0, widths[j]), :],
          in_sems.at[j % _NBUF])

    def out_copy(j):
      if rem and j == n_tiles - 1:
        return pltpu.make_async_copy(
            o_tail, out_hbm.at[:, pl.ds(j * vt, rem)], tail_out_sem)
      return pltpu.make_async_copy(
          o_bufs.at[j % _OBUF], out_hbm.at[:, pl.ds(j * vt, vt)],
          out_sems.at[j % _OBUF])

    for j in range(min(_NBUF, n_tiles)):
      in_copy(j).start()

    enc = enc_ref[...]
    mean_v = jnp.mean(enc, axis=1)
    h = jnp.tanh(
        jnp.dot(mean_v, v2h_W_ref[...], preferred_element_type=jnp.float32)
        + v2h_b_ref[...])
    e_proj = jnp.dot(enc.reshape(b * f, h_dim), att_We_ref[...],
                     preferred_element_type=jnp.float32).reshape(b, f, h_dim)

    att_Wh = att_Wh_ref[...]
    att_v = att_v_ref[...]
    W_ih = W_ih_ref[...]
    W_hh = W_hh_ref[...]
    b_ih = b_ih_ref[...]
    b_hh = b_hh_ref[...]

    gi_emb = jnp.dot(emb_ref[...].reshape(t_steps * b, h_dim), W_ih[:h_dim],
                     preferred_element_type=jnp.float32) + b_ih

    for i in range(t_steps):
      hw = jnp.dot(h, att_Wh, preferred_element_type=jnp.float32)
      tt = jnp.tanh(hw[:, None, :] + e_proj)
      scores = jnp.sum(tt * att_v[None, None, :], axis=-1)
      e = jnp.exp(scores)
      probs = e / jnp.sum(e, axis=-1, keepdims=True)
      ctx = jnp.sum(probs[:, :, None] * enc, axis=1)

      gi = (gi_emb[i * b:(i + 1) * b]
            + jnp.dot(ctx, W_ih[h_dim:], preferred_element_type=jnp.float32))
      gh = jnp.dot(h, W_hh, preferred_element_type=jnp.float32) + b_hh
      r = jax.nn.sigmoid(gi[:, :h_dim] + gh[:, :h_dim])
      z = jax.nn.sigmoid(gi[:, h_dim:2 * h_dim] + gh[:, h_dim:2 * h_dim])
      n = jnp.tanh(gi[:, 2 * h_dim:] + r * gh[:, 2 * h_dim:])
      h = (1.0 - z) * n + z * h

      hid_ref[i] = h
      attn_ref[i] = probs

    h16 = hid_ref[...].reshape(n_rows, h_dim).astype(jnp.bfloat16)
    for j in range(n_tiles):
      is_tail = rem and j == n_tiles - 1
      wt = widths[j]
      in_copy(j).wait()
      if j >= _OBUF:
        out_copy(j - _OBUF).wait()
      w16 = w_bufs[j % _NBUF, :wt, :].astype(jnp.bfloat16)
      acc = lax.dot_general(h16, w16, (((1,), (1,)), ((), ())),
                            preferred_element_type=jnp.float32)
      res = acc + cb_ref[:, j * vt:j * vt + wt]
      if is_tail:
        o_tail[...] = res
      else:
        o_bufs[j % _OBUF] = res
      out_copy(j).start()
      if j + _NBUF < n_tiles:
        in_copy(j + _NBUF).start()
    for j in range(max(n_tiles - _OBUF, 0), n_tiles):
      out_copy(j).wait()

  return pl.pallas_call(
      body,
      in_specs=[pl.BlockSpec(memory_space=pltpu.VMEM)] * 12
      + [pl.BlockSpec(memory_space=pl.ANY)],
      out_specs=(
          pl.BlockSpec(memory_space=pltpu.VMEM),
          pl.BlockSpec(memory_space=pltpu.VMEM),
          pl.BlockSpec(memory_space=pl.ANY),
      ),
      out_shape=(
          jax.ShapeDtypeStruct((t_steps, b, h_dim), jnp.float32),
          jax.ShapeDtypeStruct((t_steps, b, f), jnp.float32),
          jax.ShapeDtypeStruct((n_rows, v), jnp.float32),
      ),
      scratch_shapes=[
          pltpu.VMEM((_NBUF, vt, h_dim), jnp.float32),
          pltpu.VMEM((_OBUF, n_rows, vt), jnp.float32),
          pltpu.VMEM((n_rows, rem if rem else 128), jnp.float32),
          pltpu.SemaphoreType.DMA((_NBUF,)),
          pltpu.SemaphoreType.DMA((_OBUF,)),
          pltpu.SemaphoreType.DMA,
      ],
  )


def kernel(input_ids, encoder_hidden_states, embedding, v2h_W, v2h_b,
           att_Wh, att_We, att_v, W_ih, W_hh, b_ih, b_hh, cls_W, cls_b):
  b, t = input_ids.shape
  _, f, h_dim = encoder_hidden_states.shape
  v = cls_W.shape[1]

  flat_ids = input_ids.T.reshape(-1).astype(jnp.int32)
  rows_tb = _make_sc_gather(t * b, h_dim)(embedding, flat_ids)
  emb_tbh = rows_tb.reshape(t, b, h_dim)
  sentence_embs = jnp.transpose(emb_tbh, (1, 0, 2))

  hid_tbh, attn_tbf, logits_flat = _make_fused(t, b, f, h_dim, v, 2048)(
      emb_tbh, encoder_hidden_states, v2h_W, v2h_b, att_Wh, att_We, att_v,
      W_ih, W_hh, b_ih, b_hh, cls_b.reshape(1, v), cls_W.T)

  out_hidden = jnp.transpose(hid_tbh, (1, 0, 2))
  out_attn = jnp.transpose(attn_tbf, (1, 2, 0))
  out_logits = jnp.transpose(logits_flat.reshape(t, b, v), (1, 0, 2))

  return out_hidden, out_attn, out_logits, sentence_embs
